# baseline (speedup 1.0000x reference)
import jax
import jax.numpy as jnp
from jax.experimental import pallas as pl

K_TOP_ = 10
TILE_C = 2048
N_CAND = 100000
N_PAD = 100352


def _matmul_body(x_ref, c_ref, o_ref):
    o_ref[...] = jax.lax.dot_general(
        x_ref[...], c_ref[...],
        dimension_numbers=(((1,), (1,)), ((), ())),
        preferred_element_type=jnp.float32,
    )


def kernel(inputs, candidates, ids):
    B, D = inputs.shape
    n_tiles = N_PAD // TILE_C
    cand_pad = jnp.pad(candidates, ((0, N_PAD - N_CAND), (0, 0)))
    scores = pl.pallas_call(
        _matmul_body,
        grid=(n_tiles,),
        in_specs=[
            pl.BlockSpec((B, D), lambda t: (0, 0)),
            pl.BlockSpec((TILE_C, D), lambda t: (t, 0)),
        ],
        out_specs=pl.BlockSpec((B, TILE_C), lambda t: (0, t)),
        out_shape=jax.ShapeDtypeStruct((B, N_PAD), jnp.float32),
    )(inputs, cand_pad)
    scores = scores[:, :N_CAND]
    top_scores, top_idx = jax.lax.top_k(scores, K_TOP_)
    top_ids = jnp.take(ids, top_idx, axis=0)
    return top_scores, top_ids


# speedup vs baseline: 5.8016x; 5.8016x over previous
import jax
import jax.numpy as jnp
from jax.experimental import pallas as pl

K_TOP_ = 10
TILE_C = 2048
SUB = 16
LANES = 128
N_CAND = 100000
N_TILES = 49
N_PAD = N_TILES * TILE_C
N_CHUNK = N_TILES * LANES
P2_ROWS = 128
P4_ROWS = 32
NEG_INF = float("-inf")
BIG = 2**30


def _p1_body(x_ref, c_ref, m_ref):
    t = pl.program_id(0)
    s = jax.lax.dot_general(
        x_ref[...], c_ref[...],
        dimension_numbers=(((1,), (1,)), ((), ())),
        preferred_element_type=jnp.float32,
    )
    lane = jax.lax.broadcasted_iota(jnp.int32, (1, LANES), 1)
    acc = jnp.full((s.shape[0], LANES), NEG_INF, dtype=jnp.float32)
    for sub in range(SUB):
        slab = s[:, sub * LANES:(sub + 1) * LANES]
        gidx = t * TILE_C + sub * LANES + lane
        slab = jnp.where(gidx < N_CAND, slab, NEG_INF)
        acc = jnp.maximum(acc, slab)
    m_ref[...] = acc


def _p2_body(m_ref, o_ref):
    m = m_ref[...]
    iota = jax.lax.broadcasted_iota(jnp.int32, m.shape, 1)
    cols = []
    for _ in range(K_TOP_):
        mx = jnp.max(m, axis=1, keepdims=True)
        idx = jnp.min(jnp.where(m == mx, iota, BIG), axis=1, keepdims=True)
        cols.append(idx)
        m = jnp.where(iota == idx, NEG_INF, m)
    o_ref[...] = jnp.concatenate(cols, axis=1)


def _p4_body(x_ref, g_ref, ix_ref, os_ref, oi_ref):
    r = P4_ROWS
    full = jax.lax.dot_general(
        x_ref[...], g_ref[...],
        dimension_numbers=(((1,), (1,)), ((), ())),
        preferred_element_type=jnp.float32,
    )
    cube = full.reshape(r, r, K_TOP_ * SUB)
    onehot3 = (jax.lax.broadcasted_iota(jnp.int32, (r, r, K_TOP_ * SUB), 0)
               == jax.lax.broadcasted_iota(jnp.int32, (r, r, K_TOP_ * SUB), 1))
    sel = jnp.max(jnp.where(onehot3, cube, NEG_INF), axis=1)
    ixf = ix_ref[...]
    sel = jnp.where(ixf < N_CAND, sel, NEG_INF)
    iota = jax.lax.broadcasted_iota(jnp.int32, sel.shape, 1)
    svals, sids = [], []
    for _ in range(K_TOP_):
        mx = jnp.max(sel, axis=1, keepdims=True)
        pos = jnp.min(jnp.where(sel == mx, iota, BIG), axis=1, keepdims=True)
        cid = jnp.sum(jnp.where(iota == pos, ixf, 0), axis=1, keepdims=True)
        svals.append(mx)
        sids.append(cid)
        sel = jnp.where(iota == pos, NEG_INF, sel)
    os_ref[...] = jnp.concatenate(svals, axis=1)
    oi_ref[...] = jnp.concatenate(sids, axis=1)


def kernel(inputs, candidates, ids):
    B, D = inputs.shape
    cand_pad = jnp.pad(candidates, ((0, N_PAD - N_CAND), (0, 0)))

    chunk_max = pl.pallas_call(
        _p1_body,
        grid=(N_TILES,),
        in_specs=[
            pl.BlockSpec((B, D), lambda t: (0, 0)),
            pl.BlockSpec((TILE_C, D), lambda t: (t, 0)),
        ],
        out_specs=pl.BlockSpec((B, LANES), lambda t: (0, t)),
        out_shape=jax.ShapeDtypeStruct((B, N_CHUNK), jnp.float32),
    )(inputs, cand_pad)

    chunk_idx = pl.pallas_call(
        _p2_body,
        grid=(B // P2_ROWS,),
        in_specs=[pl.BlockSpec((P2_ROWS, N_CHUNK), lambda r: (r, 0))],
        out_specs=pl.BlockSpec((P2_ROWS, K_TOP_), lambda r: (r, 0)),
        out_shape=jax.ShapeDtypeStruct((B, K_TOP_), jnp.int32),
    )(chunk_max)

    sub = jnp.arange(SUB, dtype=jnp.int32)
    base = (chunk_idx // LANES) * TILE_C + (chunk_idx % LANES)
    idx_full = (base[:, :, None] + LANES * sub[None, None, :]).reshape(
        B, K_TOP_ * SUB)
    gather_idx = jnp.minimum(idx_full, N_CAND - 1)

    g = jnp.take(candidates, gather_idx.reshape(-1), axis=0)

    top_scores, top_idx = pl.pallas_call(
        _p4_body,
        grid=(B // P4_ROWS,),
        in_specs=[
            pl.BlockSpec((P4_ROWS, D), lambda r: (r, 0)),
            pl.BlockSpec((P4_ROWS * K_TOP_ * SUB, D), lambda r: (r, 0)),
            pl.BlockSpec((P4_ROWS, K_TOP_ * SUB), lambda r: (r, 0)),
        ],
        out_specs=[
            pl.BlockSpec((P4_ROWS, K_TOP_), lambda r: (r, 0)),
            pl.BlockSpec((P4_ROWS, K_TOP_), lambda r: (r, 0)),
        ],
        out_shape=[
            jax.ShapeDtypeStruct((B, K_TOP_), jnp.float32),
            jax.ShapeDtypeStruct((B, K_TOP_), jnp.int32),
        ],
    )(inputs, g, idx_full)

    top_ids = jnp.take(ids, top_idx, axis=0)
    return top_scores, top_ids


# speedup vs baseline: 6.8677x; 1.1838x over previous
import functools

import jax
import jax.numpy as jnp
from jax import lax
from jax.experimental import pallas as pl
from jax.experimental.pallas import tpu as pltpu
from jax.experimental.pallas import tpu_sc as plsc

K_TOP_ = 10
TILE_C = 2048
SUB = 16
LANES = 128
N_CAND = 100000
N_TILES = 49
N_PAD = N_TILES * TILE_C
N_CHUNK = N_TILES * LANES
N_REAL_CHUNK = N_CAND // SUB
P2_ROWS = 128
P4_ROWS = 32
SEL = K_TOP_ * SUB
NEG_INF = float("-inf")
BIG = 2**30


def _p1_body(x_ref, c_ref, m_ref):
    t = pl.program_id(0)
    s = jax.lax.dot_general(
        x_ref[...], c_ref[...],
        dimension_numbers=(((1,), (1,)), ((), ())),
        preferred_element_type=jnp.float32,
    )
    acc = s[:, :LANES]
    for u in range(1, SUB):
        acc = jnp.maximum(acc, s[:, u * LANES:(u + 1) * LANES])
    lane = jax.lax.broadcasted_iota(jnp.int32, (1, LANES), 1)
    acc = jnp.where(t * LANES + lane < N_REAL_CHUNK, acc, NEG_INF)
    m_ref[...] = acc


def _p2_body(m_ref, o_ref):
    m = m_ref[...]
    iota = jax.lax.broadcasted_iota(jnp.int32, m.shape, 1)
    cols = []
    for _ in range(K_TOP_):
        mx = jnp.max(m, axis=1, keepdims=True)
        idx = jnp.min(jnp.where(m == mx, iota, BIG), axis=1, keepdims=True)
        cols.append(idx)
        m = jnp.where(iota == idx, NEG_INF, m)
    o_ref[...] = jnp.concatenate(cols, axis=1)


def _p4_body(x_ref, g_ref, ix_ref, os_ref, oi_ref):
    r = P4_ROWS
    full = jax.lax.dot_general(
        x_ref[...], g_ref[...],
        dimension_numbers=(((1,), (1,)), ((), ())),
        preferred_element_type=jnp.float32,
    )
    cube = full.reshape(r, r, SEL)
    onehot3 = (jax.lax.broadcasted_iota(jnp.int32, (r, r, SEL), 0)
               == jax.lax.broadcasted_iota(jnp.int32, (r, r, SEL), 1))
    sel = jnp.max(jnp.where(onehot3, cube, NEG_INF), axis=1)
    ixf = ix_ref[...]
    iota = jax.lax.broadcasted_iota(jnp.int32, sel.shape, 1)
    svals, sids = [], []
    for _ in range(K_TOP_):
        mx = jnp.max(sel, axis=1, keepdims=True)
        pos = jnp.min(jnp.where(sel == mx, iota, BIG), axis=1, keepdims=True)
        cid = jnp.sum(jnp.where(iota == pos, ixf, 0), axis=1, keepdims=True)
        svals.append(mx)
        sids.append(cid)
        sel = jnp.where(iota == pos, NEG_INF, sel)
    os_ref[...] = jnp.concatenate(svals, axis=1)
    oi_ref[...] = jnp.concatenate(sids, axis=1)


_GATHER_B = 1024 * K_TOP_
_ROW_W = SUB * 32
_NW = 32
_PER_W = _GATHER_B // _NW
_CHUNK_G = 80


@functools.partial(
    pl.kernel,
    mesh=plsc.VectorSubcoreMesh(core_axis_name="c", subcore_axis_name="s"),
    out_type=jax.ShapeDtypeStruct((_GATHER_B, _ROW_W), jnp.float32),
    scratch_types=[
        pltpu.VMEM((_CHUNK_G,), jnp.int32),
        pltpu.VMEM((_CHUNK_G, _ROW_W), jnp.float32),
        pltpu.SemaphoreType.DMA,
    ],
)
def _sc_gather(table_hbm, idx_hbm, out_hbm, idx_v, rows_v, sem):
    wid = lax.axis_index("s") * 2 + lax.axis_index("c")
    for ci in range(_PER_W // _CHUNK_G):
        base = wid * _PER_W + ci * _CHUNK_G
        pltpu.sync_copy(idx_hbm.at[pl.ds(base, _CHUNK_G)], idx_v)
        pltpu.async_copy(table_hbm.at[idx_v], rows_v, sem).wait()
        pltpu.sync_copy(rows_v, out_hbm.at[pl.ds(base, _CHUNK_G)])


def kernel(inputs, candidates, ids):
    B, D = inputs.shape
    cand_pad = jnp.pad(candidates, ((0, N_PAD - N_CAND), (0, 0)))
    cand_perm = (cand_pad.reshape(N_TILES, LANES, SUB, D)
                 .transpose(0, 2, 1, 3).reshape(N_PAD, D))
    table = cand_pad.reshape(N_CHUNK, _ROW_W)

    chunk_max = pl.pallas_call(
        _p1_body,
        grid=(N_TILES,),
        in_specs=[
            pl.BlockSpec((B, D), lambda t: (0, 0)),
            pl.BlockSpec((TILE_C, D), lambda t: (t, 0)),
        ],
        out_specs=pl.BlockSpec((B, LANES), lambda t: (0, t)),
        out_shape=jax.ShapeDtypeStruct((B, N_CHUNK), jnp.float32),
    )(inputs, cand_perm)

    chunk_idx = pl.pallas_call(
        _p2_body,
        grid=(B // P2_ROWS,),
        in_specs=[pl.BlockSpec((P2_ROWS, N_CHUNK), lambda r: (r, 0))],
        out_specs=pl.BlockSpec((P2_ROWS, K_TOP_), lambda r: (r, 0)),
        out_shape=jax.ShapeDtypeStruct((B, K_TOP_), jnp.int32),
    )(chunk_max)

    g = _sc_gather(table, chunk_idx.reshape(-1))
    g = g.reshape(B * SEL, D)

    sub = jnp.arange(SUB, dtype=jnp.int32)
    idx_full = (chunk_idx[:, :, None] * SUB + sub[None, None, :]).reshape(
        B, SEL)

    top_scores, top_idx = pl.pallas_call(
        _p4_body,
        grid=(B // P4_ROWS,),
        in_specs=[
            pl.BlockSpec((P4_ROWS, D), lambda r: (r, 0)),
            pl.BlockSpec((P4_ROWS * SEL, D), lambda r: (r, 0)),
            pl.BlockSpec((P4_ROWS, SEL), lambda r: (r, 0)),
        ],
        out_specs=[
            pl.BlockSpec((P4_ROWS, K_TOP_), lambda r: (r, 0)),
            pl.BlockSpec((P4_ROWS, K_TOP_), lambda r: (r, 0)),
        ],
        out_shape=[
            jax.ShapeDtypeStruct((B, K_TOP_), jnp.float32),
            jax.ShapeDtypeStruct((B, K_TOP_), jnp.int32),
        ],
    )(inputs, g, idx_full)

    top_ids = jnp.take(ids, top_idx, axis=0)
    return top_scores, top_ids


# speedup vs baseline: 8.6378x; 1.2578x over previous
import functools

import jax
import jax.numpy as jnp
from jax import lax
from jax.experimental import pallas as pl
from jax.experimental.pallas import tpu as pltpu
from jax.experimental.pallas import tpu_sc as plsc

K_TOP_ = 10
TILE_C = 2048
SUB = 16
LANES = 128
N_CAND = 100000
N_TILES = 49
N_CHUNK = N_TILES * LANES
P2_ROWS = 128
P4_ROWS = 32
SEL = K_TOP_ * SUB
ROW_W = SUB * 32
NEG_INF = float("-inf")
BIG = 2**30


def _p1_body(x_ref, c_ref, m_ref, tb_ref):
    t = pl.program_id(0)
    c = c_ref[...]
    s = jax.lax.dot_general(
        x_ref[...], c,
        dimension_numbers=(((1,), (1,)), ((), ())),
        preferred_element_type=jnp.float32,
    )
    lane = jax.lax.broadcasted_iota(jnp.int32, (1, LANES), 1)
    acc = None
    for u in range(SUB):
        slab = s[:, u * LANES:(u + 1) * LANES]
        gidx = t * TILE_C + u * LANES + lane
        slab = jnp.where(gidx < N_CAND, slab, NEG_INF)
        acc = slab if acc is None else jnp.maximum(acc, slab)
    m_ref[...] = acc
    tb_ref[...] = jnp.concatenate(
        [c[u * LANES:(u + 1) * LANES, :] for u in range(SUB)], axis=1)


def _p2_body(m_ref, o_ref):
    m = m_ref[...]
    iota = jax.lax.broadcasted_iota(jnp.int32, m.shape, 1)
    cols = []
    for _ in range(K_TOP_):
        mx = jnp.max(m, axis=1, keepdims=True)
        idx = jnp.min(jnp.where(m == mx, iota, BIG), axis=1, keepdims=True)
        cols.append(idx)
        m = jnp.where(iota == idx, NEG_INF, m)
    o_ref[...] = jnp.concatenate(cols, axis=1)


def _p4_body(x_ref, g_ref, ix_ref, os_ref, oi_ref):
    r = P4_ROWS
    gb = g_ref[...]
    x = x_ref[...]
    onehot3 = (jax.lax.broadcasted_iota(jnp.int32, (r, r, K_TOP_), 0)
               == jax.lax.broadcasted_iota(jnp.int32, (r, r, K_TOP_), 1))
    sel_parts = []
    for u in range(SUB):
        gu = gb[:, u * 32:(u + 1) * 32]
        full_u = jax.lax.dot_general(
            x, gu,
            dimension_numbers=(((1,), (1,)), ((), ())),
            preferred_element_type=jnp.float32,
        )
        cube = full_u.reshape(r, r, K_TOP_)
        sel_parts.append(jnp.max(jnp.where(onehot3, cube, NEG_INF), axis=1))
    sel = jnp.concatenate(sel_parts, axis=1)
    ixf = ix_ref[...]
    sel = jnp.where(ixf < N_CAND, sel, NEG_INF)
    iota = jax.lax.broadcasted_iota(jnp.int32, sel.shape, 1)
    svals, sids = [], []
    for _ in range(K_TOP_):
        mx = jnp.max(sel, axis=1, keepdims=True)
        pos = jnp.min(jnp.where(sel == mx, iota, BIG), axis=1, keepdims=True)
        cid = jnp.sum(jnp.where(iota == pos, ixf, 0), axis=1, keepdims=True)
        svals.append(mx)
        sids.append(cid)
        sel = jnp.where(iota == pos, NEG_INF, sel)
    os_ref[...] = jnp.concatenate(svals, axis=1)
    oi_ref[...] = jnp.concatenate(sids, axis=1)


_GATHER_B = 1024 * K_TOP_
_NW = 32
_PER_W = _GATHER_B // _NW
_CHUNK_G = 80


@functools.partial(
    pl.kernel,
    mesh=plsc.VectorSubcoreMesh(core_axis_name="c", subcore_axis_name="s"),
    out_type=jax.ShapeDtypeStruct((_GATHER_B, ROW_W), jnp.float32),
    scratch_types=[
        pltpu.VMEM((_CHUNK_G,), jnp.int32),
        pltpu.VMEM((_CHUNK_G, ROW_W), jnp.float32),
        pltpu.SemaphoreType.DMA,
    ],
)
def _sc_gather(table_hbm, idx_hbm, out_hbm, idx_v, rows_v, sem):
    wid = lax.axis_index("s") * 2 + lax.axis_index("c")
    for ci in range(_PER_W // _CHUNK_G):
        base = wid * _PER_W + ci * _CHUNK_G
        pltpu.sync_copy(idx_hbm.at[pl.ds(base, _CHUNK_G)], idx_v)
        pltpu.async_copy(table_hbm.at[idx_v], rows_v, sem).wait()
        pltpu.sync_copy(rows_v, out_hbm.at[pl.ds(base, _CHUNK_G)])


def kernel(inputs, candidates, ids):
    B, D = inputs.shape

    chunk_max, table = pl.pallas_call(
        _p1_body,
        grid=(N_TILES,),
        in_specs=[
            pl.BlockSpec((B, D), lambda t: (0, 0)),
            pl.BlockSpec((TILE_C, D), lambda t: (t, 0)),
        ],
        out_specs=[
            pl.BlockSpec((B, LANES), lambda t: (0, t)),
            pl.BlockSpec((LANES, ROW_W), lambda t: (t, 0)),
        ],
        out_shape=[
            jax.ShapeDtypeStruct((B, N_CHUNK), jnp.float32),
            jax.ShapeDtypeStruct((N_CHUNK, ROW_W), jnp.float32),
        ],
    )(inputs, candidates)

    chunk_idx = pl.pallas_call(
        _p2_body,
        grid=(B // P2_ROWS,),
        in_specs=[pl.BlockSpec((P2_ROWS, N_CHUNK), lambda r: (r, 0))],
        out_specs=pl.BlockSpec((P2_ROWS, K_TOP_), lambda r: (r, 0)),
        out_shape=jax.ShapeDtypeStruct((B, K_TOP_), jnp.int32),
    )(chunk_max)

    g = _sc_gather(table, chunk_idx.reshape(-1))

    sub = jnp.arange(SUB, dtype=jnp.int32)
    base = (chunk_idx // LANES) * TILE_C + (chunk_idx % LANES)
    idx_full = (base[:, None, :] + LANES * sub[None, :, None]).reshape(B, SEL)

    top_scores, top_idx = pl.pallas_call(
        _p4_body,
        grid=(B // P4_ROWS,),
        in_specs=[
            pl.BlockSpec((P4_ROWS, D), lambda r: (r, 0)),
            pl.BlockSpec((P4_ROWS * K_TOP_, ROW_W), lambda r: (r, 0)),
            pl.BlockSpec((P4_ROWS, SEL), lambda r: (r, 0)),
        ],
        out_specs=[
            pl.BlockSpec((P4_ROWS, K_TOP_), lambda r: (r, 0)),
            pl.BlockSpec((P4_ROWS, K_TOP_), lambda r: (r, 0)),
        ],
        out_shape=[
            jax.ShapeDtypeStruct((B, K_TOP_), jnp.float32),
            jax.ShapeDtypeStruct((B, K_TOP_), jnp.int32),
        ],
    )(inputs, g, idx_full)

    top_ids = jnp.take(ids, top_idx, axis=0)
    return top_scores, top_ids


# speedup vs baseline: 10.5397x; 1.2202x over previous
import functools

import jax
import jax.numpy as jnp
from jax import lax
from jax.experimental import pallas as pl
from jax.experimental.pallas import tpu as pltpu
from jax.experimental.pallas import tpu_sc as plsc

K_TOP_ = 10
TILE_C = 2048
SUB = 16
LANES = 128
N_CAND = 100000
N_TILES = 49
N_CHUNK = N_TILES * LANES
P2_ROWS = 128
P4_ROWS = 256
SEL = K_TOP_ * SUB
ROW_W = SUB * 32
NEG_INF = float("-inf")
BIG = 2**30


def _p1_body(x_ref, c_ref, m_ref, tb_ref):
    t = pl.program_id(0)
    c = c_ref[...]
    s = jax.lax.dot_general(
        x_ref[...], c,
        dimension_numbers=(((1,), (1,)), ((), ())),
        preferred_element_type=jnp.float32,
    )
    lane = jax.lax.broadcasted_iota(jnp.int32, (1, LANES), 1)
    acc = None
    for u in range(SUB):
        slab = s[:, u * LANES:(u + 1) * LANES]
        gidx = t * TILE_C + u * LANES + lane
        slab = jnp.where(gidx < N_CAND, slab, NEG_INF)
        acc = slab if acc is None else jnp.maximum(acc, slab)
    m_ref[...] = acc
    tb_ref[...] = jnp.concatenate(
        [c[u * LANES:(u + 1) * LANES, :] for u in range(SUB)], axis=1)


def _p2_body(m_ref, o_ref):
    m = m_ref[...]
    iota = jax.lax.broadcasted_iota(jnp.int32, m.shape, 1)
    cols = []
    for _ in range(K_TOP_):
        mx = jnp.max(m, axis=1, keepdims=True)
        idx = jnp.min(jnp.where(m == mx, iota, BIG), axis=1, keepdims=True)
        cols.append(idx)
        m = jnp.where(iota == idx, NEG_INF, m)
    o_ref[...] = jnp.concatenate(cols, axis=1)


def _p4_body(x_ref, g_ref, ix_ref, os_ref, oi_ref):
    r = P4_ROWS
    gb = g_ref[...]
    x = x_ref[...]
    onehot3 = (jax.lax.broadcasted_iota(jnp.int32, (r, r, K_TOP_), 0)
               == jax.lax.broadcasted_iota(jnp.int32, (r, r, K_TOP_), 1))
    sel_parts = []
    for u in range(SUB):
        gu = gb[:, u * 32:(u + 1) * 32]
        full_u = jax.lax.dot_general(
            x, gu,
            dimension_numbers=(((1,), (1,)), ((), ())),
            preferred_element_type=jnp.float32,
        )
        cube = full_u.reshape(r, r, K_TOP_)
        sel_parts.append(jnp.max(jnp.where(onehot3, cube, NEG_INF), axis=1))
    sel = jnp.concatenate(sel_parts, axis=1)
    ixf = ix_ref[...]
    sel = jnp.where(ixf < N_CAND, sel, NEG_INF)
    iota = jax.lax.broadcasted_iota(jnp.int32, sel.shape, 1)
    svals, sids = [], []
    for _ in range(K_TOP_):
        mx = jnp.max(sel, axis=1, keepdims=True)
        pos = jnp.min(jnp.where(sel == mx, iota, BIG), axis=1, keepdims=True)
        cid = jnp.sum(jnp.where(iota == pos, ixf, 0), axis=1, keepdims=True)
        svals.append(mx)
        sids.append(cid)
        sel = jnp.where(iota == pos, NEG_INF, sel)
    os_ref[...] = jnp.concatenate(svals, axis=1)
    oi_ref[...] = jnp.concatenate(sids, axis=1)


_GATHER_B = 1024 * K_TOP_
_NW = 32
_PER_W = _GATHER_B // _NW
_CHUNK_G = 80


@functools.partial(
    pl.kernel,
    mesh=plsc.VectorSubcoreMesh(core_axis_name="c", subcore_axis_name="s"),
    out_type=jax.ShapeDtypeStruct((_GATHER_B, ROW_W), jnp.float32),
    scratch_types=[
        pltpu.VMEM((_CHUNK_G,), jnp.int32),
        pltpu.VMEM((_CHUNK_G, ROW_W), jnp.float32),
        pltpu.SemaphoreType.DMA,
    ],
)
def _sc_gather(table_hbm, idx_hbm, out_hbm, idx_v, rows_v, sem):
    wid = lax.axis_index("s") * 2 + lax.axis_index("c")
    for ci in range(_PER_W // _CHUNK_G):
        base = wid * _PER_W + ci * _CHUNK_G
        pltpu.sync_copy(idx_hbm.at[pl.ds(base, _CHUNK_G)], idx_v)
        pltpu.async_copy(table_hbm.at[idx_v], rows_v, sem).wait()
        pltpu.sync_copy(rows_v, out_hbm.at[pl.ds(base, _CHUNK_G)])


def kernel(inputs, candidates, ids):
    B, D = inputs.shape

    chunk_max, table = pl.pallas_call(
        _p1_body,
        grid=(N_TILES,),
        in_specs=[
            pl.BlockSpec((B, D), lambda t: (0, 0)),
            pl.BlockSpec((TILE_C, D), lambda t: (t, 0)),
        ],
        out_specs=[
            pl.BlockSpec((B, LANES), lambda t: (0, t)),
            pl.BlockSpec((LANES, ROW_W), lambda t: (t, 0)),
        ],
        out_shape=[
            jax.ShapeDtypeStruct((B, N_CHUNK), jnp.float32),
            jax.ShapeDtypeStruct((N_CHUNK, ROW_W), jnp.float32),
        ],
    )(inputs, candidates)

    chunk_idx = pl.pallas_call(
        _p2_body,
        grid=(B // P2_ROWS,),
        in_specs=[pl.BlockSpec((P2_ROWS, N_CHUNK), lambda r: (r, 0))],
        out_specs=pl.BlockSpec((P2_ROWS, K_TOP_), lambda r: (r, 0)),
        out_shape=jax.ShapeDtypeStruct((B, K_TOP_), jnp.int32),
    )(chunk_max)

    g = _sc_gather(table, chunk_idx.reshape(-1))

    sub = jnp.arange(SUB, dtype=jnp.int32)
    base = (chunk_idx // LANES) * TILE_C + (chunk_idx % LANES)
    idx_full = (base[:, None, :] + LANES * sub[None, :, None]).reshape(B, SEL)

    top_scores, top_idx = pl.pallas_call(
        _p4_body,
        grid=(B // P4_ROWS,),
        in_specs=[
            pl.BlockSpec((P4_ROWS, D), lambda r: (r, 0)),
            pl.BlockSpec((P4_ROWS * K_TOP_, ROW_W), lambda r: (r, 0)),
            pl.BlockSpec((P4_ROWS, SEL), lambda r: (r, 0)),
        ],
        out_specs=[
            pl.BlockSpec((P4_ROWS, K_TOP_), lambda r: (r, 0)),
            pl.BlockSpec((P4_ROWS, K_TOP_), lambda r: (r, 0)),
        ],
        out_shape=[
            jax.ShapeDtypeStruct((B, K_TOP_), jnp.float32),
            jax.ShapeDtypeStruct((B, K_TOP_), jnp.int32),
        ],
    )(inputs, g, idx_full)

    top_ids = jnp.take(ids, top_idx, axis=0)
    return top_scores, top_ids


# speedup vs baseline: 10.6582x; 1.0112x over previous
import functools

import jax
import jax.numpy as jnp
from jax import lax
from jax.experimental import pallas as pl
from jax.experimental.pallas import tpu as pltpu
from jax.experimental.pallas import tpu_sc as plsc

K_TOP_ = 10
TILE_C = 2048
SUB = 16
LANES = 128
N_CAND = 100000
N_TILES = 49
N_CHUNK = N_TILES * LANES
P2_ROWS = 128
P4_ROWS = 256
SEL = K_TOP_ * SUB
ROW_W = SUB * 32
NEG_INF = float("-inf")
BIG = 2**30


def _p1_body(x_ref, c_ref, m_ref, tb_ref):
    t = pl.program_id(0)
    c = c_ref[...]
    s = jax.lax.dot_general(
        x_ref[...], c,
        dimension_numbers=(((1,), (1,)), ((), ())),
        preferred_element_type=jnp.float32,
    )
    lane = jax.lax.broadcasted_iota(jnp.int32, (1, LANES), 1)
    acc = None
    for u in range(SUB):
        slab = s[:, u * LANES:(u + 1) * LANES]
        gidx = t * TILE_C + u * LANES + lane
        slab = jnp.where(gidx < N_CAND, slab, NEG_INF)
        acc = slab if acc is None else jnp.maximum(acc, slab)
    m_ref[...] = acc
    tb_ref[...] = jnp.concatenate(
        [c[u * LANES:(u + 1) * LANES, :] for u in range(SUB)], axis=1)


def _p2_body(m_ref, o_ref):
    m = m_ref[...]
    iota = jax.lax.broadcasted_iota(jnp.int32, m.shape, 1)
    cols = []
    for _ in range(K_TOP_):
        mx = jnp.max(m, axis=1, keepdims=True)
        idx = jnp.min(jnp.where(m == mx, iota, BIG), axis=1, keepdims=True)
        cols.append(idx)
        m = jnp.where(iota == idx, NEG_INF, m)
    o_ref[...] = jnp.concatenate(cols, axis=1)


def _p4_body(x_ref, g_ref, ix_ref, os_ref, oi_ref):
    r = P4_ROWS
    gb = g_ref[...]
    x = x_ref[...]
    onehot3 = (jax.lax.broadcasted_iota(jnp.int32, (r, r, K_TOP_), 0)
               == jax.lax.broadcasted_iota(jnp.int32, (r, r, K_TOP_), 1))
    sel_parts = []
    for u in range(SUB):
        gu = gb[:, u * 32:(u + 1) * 32]
        full_u = jax.lax.dot_general(
            x, gu,
            dimension_numbers=(((1,), (1,)), ((), ())),
            preferred_element_type=jnp.float32,
        )
        cube = full_u.reshape(r, r, K_TOP_)
        sel_parts.append(jnp.max(jnp.where(onehot3, cube, NEG_INF), axis=1))
    sel = jnp.concatenate(sel_parts, axis=1)
    ixf = ix_ref[...]
    sel = jnp.where(ixf < N_CAND, sel, NEG_INF)
    iota = jax.lax.broadcasted_iota(jnp.int32, sel.shape, 1)
    svals, sids = [], []
    for _ in range(K_TOP_):
        mx = jnp.max(sel, axis=1, keepdims=True)
        pos = jnp.min(jnp.where(sel == mx, iota, BIG), axis=1, keepdims=True)
        cid = jnp.sum(jnp.where(iota == pos, ixf, 0), axis=1, keepdims=True)
        svals.append(mx)
        sids.append(cid)
        sel = jnp.where(iota == pos, NEG_INF, sel)
    os_ref[...] = jnp.concatenate(svals, axis=1)
    oi_ref[...] = jnp.concatenate(sids, axis=1)


_GATHER_B = 1024 * K_TOP_
_NW = 32
_PER_W = _GATHER_B // _NW
_CHUNK_G = 80


@functools.partial(
    pl.kernel,
    mesh=plsc.VectorSubcoreMesh(core_axis_name="c", subcore_axis_name="s"),
    out_type=jax.ShapeDtypeStruct((_GATHER_B, ROW_W), jnp.float32),
    scratch_types=[
        pltpu.VMEM((2, _CHUNK_G), jnp.int32),
        pltpu.VMEM((2, _CHUNK_G, ROW_W), jnp.float32),
        pltpu.SemaphoreType.DMA,
        pltpu.SemaphoreType.DMA,
    ],
)
def _sc_gather(table_hbm, idx_hbm, out_hbm, idx_v, rows_v, gsem, osem):
    wid = lax.axis_index("s") * 2 + lax.axis_index("c")
    n = _PER_W // _CHUNK_G

    pltpu.sync_copy(idx_hbm.at[pl.ds(wid * _PER_W, _CHUNK_G)], idx_v.at[0])
    gath_prev = pltpu.async_copy(
        table_hbm.at[idx_v.at[0]], rows_v.at[0], gsem)
    outs = []
    for ci in range(1, n):
        base = wid * _PER_W + ci * _CHUNK_G
        b = ci % 2
        if ci >= 2:
            outs[ci - 2].wait()
        pltpu.sync_copy(idx_hbm.at[pl.ds(base, _CHUNK_G)], idx_v.at[b])
        gath_next = pltpu.async_copy(
            table_hbm.at[idx_v.at[b]], rows_v.at[b], gsem)
        gath_prev.wait()
        outs.append(pltpu.async_copy(
            rows_v.at[1 - b],
            out_hbm.at[pl.ds(base - _CHUNK_G, _CHUNK_G)], osem))
        gath_prev = gath_next
    gath_prev.wait()
    if n >= 2:
        outs[n - 2].wait()
    pltpu.sync_copy(
        rows_v.at[(n - 1) % 2],
        out_hbm.at[pl.ds(wid * _PER_W + (n - 1) * _CHUNK_G, _CHUNK_G)])


def kernel(inputs, candidates, ids):
    B, D = inputs.shape

    chunk_max, table = pl.pallas_call(
        _p1_body,
        grid=(N_TILES,),
        in_specs=[
            pl.BlockSpec((B, D), lambda t: (0, 0)),
            pl.BlockSpec((TILE_C, D), lambda t: (t, 0)),
        ],
        out_specs=[
            pl.BlockSpec((B, LANES), lambda t: (0, t)),
            pl.BlockSpec((LANES, ROW_W), lambda t: (t, 0)),
        ],
        out_shape=[
            jax.ShapeDtypeStruct((B, N_CHUNK), jnp.float32),
            jax.ShapeDtypeStruct((N_CHUNK, ROW_W), jnp.float32),
        ],
    )(inputs, candidates)

    chunk_idx = pl.pallas_call(
        _p2_body,
        grid=(B // P2_ROWS,),
        in_specs=[pl.BlockSpec((P2_ROWS, N_CHUNK), lambda r: (r, 0))],
        out_specs=pl.BlockSpec((P2_ROWS, K_TOP_), lambda r: (r, 0)),
        out_shape=jax.ShapeDtypeStruct((B, K_TOP_), jnp.int32),
    )(chunk_max)

    g = _sc_gather(table, chunk_idx.reshape(-1))

    sub = jnp.arange(SUB, dtype=jnp.int32)
    base = (chunk_idx // LANES) * TILE_C + (chunk_idx % LANES)
    idx_full = (base[:, None, :] + LANES * sub[None, :, None]).reshape(B, SEL)

    top_scores, top_idx = pl.pallas_call(
        _p4_body,
        grid=(B // P4_ROWS,),
        in_specs=[
            pl.BlockSpec((P4_ROWS, D), lambda r: (r, 0)),
            pl.BlockSpec((P4_ROWS * K_TOP_, ROW_W), lambda r: (r, 0)),
            pl.BlockSpec((P4_ROWS, SEL), lambda r: (r, 0)),
        ],
        out_specs=[
            pl.BlockSpec((P4_ROWS, K_TOP_), lambda r: (r, 0)),
            pl.BlockSpec((P4_ROWS, K_TOP_), lambda r: (r, 0)),
        ],
        out_shape=[
            jax.ShapeDtypeStruct((B, K_TOP_), jnp.float32),
            jax.ShapeDtypeStruct((B, K_TOP_), jnp.int32),
        ],
    )(inputs, g, idx_full)

    top_ids = jnp.take(ids, top_idx, axis=0)
    return top_scores, top_ids
